# baseline (speedup 1.0000x reference)
import functools

import jax
import jax.numpy as jnp
from jax.experimental import pallas as pl
from jax.experimental.pallas import tpu as pltpu

N = 4096
BM = 512
BK = 600


def _bf(t):
    return t.astype(jnp.bfloat16)


def _dot0(a, b):
    return jax.lax.dot_general(a, b, (((0,), (0,)), ((), ())),
                               preferred_element_type=jnp.float32)


def _trunk_body(xt_ref, w1, b1, w2, b2, w3, b3, wh, bh, wd1, bd1, wd2, bd2,
                wd3, bd3, ht_ref, d3t_ref):
    def lin(t, w, b):
        return _dot0(w[...], t) + b[...]

    t = jnp.maximum(lin(xt_ref[...], w1, b1), 0.0)
    t = jnp.maximum(lin(t, w2, b2), 0.0)
    t = jnp.maximum(lin(t, w3, b3), 0.0)
    ht = lin(t, wh, bh)
    ht_ref[...] = ht
    t = jnp.maximum(lin(ht, wd1, bd1), 0.0)
    t = jnp.maximum(lin(t, wd2, bd2), 0.0)
    d3t = jnp.maximum(lin(t, wd3, bd3), 0.0)
    d3t_ref[...] = _bf(d3t)


def _run_trunk(xt, ws, bs):
    full = lambda a: pl.BlockSpec(a.shape, lambda i: (0,) * a.ndim)
    in_specs = [pl.BlockSpec((3000, BM), lambda i: (0, i))]
    args = []
    for w, b in zip(ws, bs):
        in_specs += [full(w), full(b)]
        args += [w, b]
    return pl.pallas_call(
        _trunk_body,
        grid=(N // BM,),
        in_specs=in_specs,
        out_specs=(pl.BlockSpec((64, BM), lambda i: (0, i)),
                   pl.BlockSpec((1024, BM), lambda i: (0, i))),
        out_shape=(jax.ShapeDtypeStruct((64, N), jnp.float32),
                   jax.ShapeDtypeStruct((1024, N), jnp.bfloat16)),
    )(xt, *args)


def _head_body(wt_ref, b_ref, d3t_ref, o_ref, *, act):
    acc = jnp.dot(_bf(wt_ref[...]), d3t_ref[...],
                  preferred_element_type=jnp.float32)
    acc = acc + b_ref[0]
    if act == "sigmoid":
        acc = jax.nn.sigmoid(acc)
    elif act == "expclip":
        acc = jnp.exp(jnp.clip(acc, -15.0, 15.0))
    elif act == "softplus":
        m = jnp.minimum(acc, 20.0) * 1.4426950408889634
        sp = 0.6931471805599453 * jnp.log2(1.0 + jnp.exp2(m))
        acc = jnp.where(acc > 20.0, acc, sp)
    o_ref[...] = acc


def _run_head(d3t, wt, bstack, head, act):
    return pl.pallas_call(
        functools.partial(_head_body, act=act),
        grid=(3000 // BK,),
        in_specs=[pl.BlockSpec((BK, 1024), lambda i: (i, 0)),
                  pl.BlockSpec((1, BK, 1), lambda i: (head, i, 0)),
                  pl.BlockSpec(d3t.shape, lambda i: (0, 0))],
        out_specs=pl.BlockSpec((BK, N), lambda i: (i, 0)),
        out_shape=jax.ShapeDtypeStruct((3000, N), jnp.float32),
    )(wt, bstack, d3t)


GB = 256
NB = N // GB


def _gcn_chain_body(act_ref, adj_ref, ht_ref, wg1, wg2, wgz, wd1, wd2, wdz,
                    z_ref, zhat_ref, adjb_ref, fa_ref, fb_ref, zb_ref):
    s = pl.program_id(0)
    r = pl.program_id(1)
    act = act_ref[0]

    def relu_maybe(t):
        return jnp.where(act != 0, jnp.maximum(t, 0.0), t)

    rows = pl.ds(r * GB, GB)

    @pl.when(s == 0)
    def _():
        ab = _bf(adj_ref[...])
        adjb_ref[rows, :] = ab
        u = _bf(_dot0(ht_ref[...], wg1[...]))
        fa_ref[rows, :] = _bf(relu_maybe(
            jnp.dot(ab, u, preferred_element_type=jnp.float32)))

    @pl.when(s == 1)
    def _():
        u = _bf(jnp.dot(fa_ref[...], wg2[...],
                        preferred_element_type=jnp.float32))
        fb_ref[rows, :] = _bf(relu_maybe(
            jnp.dot(adjb_ref[rows, :], u, preferred_element_type=jnp.float32)))

    @pl.when(s == 2)
    def _():
        u = _bf(jnp.dot(fb_ref[...], wgz[...],
                        preferred_element_type=jnp.float32))
        t = jnp.dot(adjb_ref[rows, :], u, preferred_element_type=jnp.float32)
        z_ref[...] = t
        zb_ref[rows, :] = _bf(t)

    @pl.when(s == 3)
    def _():
        u = _bf(jnp.dot(zb_ref[...], wd1[...],
                        preferred_element_type=jnp.float32))
        fb_ref[rows, :] = _bf(relu_maybe(
            jnp.dot(adjb_ref[rows, :], u, preferred_element_type=jnp.float32)))

    @pl.when(s == 4)
    def _():
        u = _bf(jnp.dot(fb_ref[...], wd2[...],
                        preferred_element_type=jnp.float32))
        fa_ref[rows, :] = _bf(relu_maybe(
            jnp.dot(adjb_ref[rows, :], u, preferred_element_type=jnp.float32)))

    @pl.when(s == 5)
    def _():
        u = _bf(jnp.dot(fa_ref[...], wdz[...],
                        preferred_element_type=jnp.float32))
        zhat_ref[...] = jnp.dot(adjb_ref[rows, :], u,
                                preferred_element_type=jnp.float32)


def _run_gcn_chain(active_s, adj, ht, p):
    full = lambda a: pl.BlockSpec(a.shape, lambda s, r: (0,) * a.ndim)
    return pl.pallas_call(
        _gcn_chain_body,
        grid=(6, NB),
        in_specs=[pl.BlockSpec(memory_space=pltpu.SMEM),
                  pl.BlockSpec((GB, N),
                               lambda s, r: (jnp.where(s == 0, r, NB - 1), 0)),
                  full(ht),
                  full(p["Wg1"]), full(p["Wg2"]), full(p["Wgz"]),
                  full(p["Wd1"]), full(p["Wd2"]), full(p["Wdz"])],
        out_specs=(
            pl.BlockSpec((GB, 16),
                         lambda s, r: (jnp.where(s == 2, r, NB - 1), 0)),
            pl.BlockSpec((GB, 64),
                         lambda s, r: (jnp.where(s == 5, r, NB - 1), 0)),
        ),
        out_shape=(jax.ShapeDtypeStruct((N, 16), jnp.float32),
                   jax.ShapeDtypeStruct((N, 64), jnp.float32)),
        scratch_shapes=[pltpu.VMEM((N, N), jnp.bfloat16),
                        pltpu.VMEM((N, 256), jnp.bfloat16),
                        pltpu.VMEM((N, 64), jnp.bfloat16),
                        pltpu.VMEM((N, 16), jnp.bfloat16)],
    )(active_s, adj, ht, p["Wg1"], p["Wg2"], p["Wgz"],
      p["Wd1"], p["Wd2"], p["Wdz"])


def _adjhat_body(zr_ref, zc_ref, o_ref):
    acc = jax.lax.dot_general(zr_ref[...], zc_ref[...],
                              (((1,), (1,)), ((), ())),
                              preferred_element_type=jnp.float32)
    o_ref[...] = jax.nn.sigmoid(acc)


def _run_adjhat(z):
    bm, bn = 1024, 4096
    return pl.pallas_call(
        _adjhat_body,
        grid=(N // bm, N // bn),
        in_specs=[pl.BlockSpec((bm, 16), lambda i, j: (i, 0)),
                  pl.BlockSpec((bn, 16), lambda i, j: (j, 0))],
        out_specs=pl.BlockSpec((bm, bn), lambda i, j: (i, j)),
        out_shape=jax.ShapeDtypeStruct((N, N), jnp.float32),
    )(z, z)


def kernel(x, adj, active, params):
    p = params
    active_s = jnp.reshape(jnp.asarray(active, jnp.int32), (1,))

    trunk_w = [p[k] for k in
               ("W_en1", "W_en2", "W_en3", "W_h", "W_de1", "W_de2", "W_de3")]
    trunk_b = [jnp.reshape(p[k], (-1, 1)) for k in
               ("b_en1", "b_en2", "b_en3", "b_h", "b_de1", "b_de2", "b_de3")]
    ht, d3t = _run_trunk(x.T, trunk_w, trunk_b)
    h = ht.T

    bstack = jnp.reshape(
        jnp.stack([p["b_xhat"], p["b_pi"], p["b_mu"], p["b_theta"]]),
        (4, 3000, 1))
    x_hat = _run_head(d3t, p["W_xhat"].T, bstack, 0, "none").T
    pi = _run_head(d3t, p["W_pi"].T, bstack, 1, "sigmoid").T
    mu = _run_head(d3t, p["W_mu"].T, bstack, 2, "expclip").T
    theta = _run_head(d3t, p["W_theta"].T, bstack, 3, "softplus").T

    z, z_hat = _run_gcn_chain(active_s, adj, ht, p)
    adj_hat = _run_adjhat(z)

    return (x_hat, pi, mu, theta, z, adj_hat, z_hat, h)


# speedup vs baseline: 1.0815x; 1.0815x over previous
import functools

import jax
import jax.numpy as jnp
from jax.experimental import pallas as pl
from jax.experimental.pallas import tpu as pltpu

N = 4096
BM = 512
BK = 600


def _bf(t):
    return t.astype(jnp.bfloat16)


def _dot0(a, b):
    return jax.lax.dot_general(a, b, (((0,), (0,)), ((), ())),
                               preferred_element_type=jnp.float32)


def _trunk_body(xt_ref, w1, b1, w2, b2, w3, b3, wh, bh, wd1, bd1, wd2, bd2,
                wd3, bd3, ht_ref, d3t_ref):
    def lin(t, w, b):
        return _dot0(w[...], t) + b[...]

    t = jnp.maximum(lin(xt_ref[...], w1, b1), 0.0)
    t = jnp.maximum(lin(t, w2, b2), 0.0)
    t = jnp.maximum(lin(t, w3, b3), 0.0)
    ht = lin(t, wh, bh)
    ht_ref[...] = ht
    t = jnp.maximum(lin(ht, wd1, bd1), 0.0)
    t = jnp.maximum(lin(t, wd2, bd2), 0.0)
    d3t = jnp.maximum(lin(t, wd3, bd3), 0.0)
    d3t_ref[...] = _bf(d3t)


def _run_trunk(xt, ws, bs):
    full = lambda a: pl.BlockSpec(a.shape, lambda i: (0,) * a.ndim)
    in_specs = [pl.BlockSpec((3000, BM), lambda i: (0, i))]
    args = []
    for w, b in zip(ws, bs):
        in_specs += [full(w), full(b)]
        args += [w, b]
    return pl.pallas_call(
        _trunk_body,
        grid=(N // BM,),
        in_specs=in_specs,
        out_specs=(pl.BlockSpec((64, BM), lambda i: (0, i)),
                   pl.BlockSpec((1024, BM), lambda i: (0, i))),
        out_shape=(jax.ShapeDtypeStruct((64, N), jnp.float32),
                   jax.ShapeDtypeStruct((1024, N), jnp.bfloat16)),
    )(xt, *args)


def _head_body(wt_ref, b_ref, d3t_ref, o_ref, *, act):
    acc = jnp.dot(_bf(wt_ref[...]), d3t_ref[...],
                  preferred_element_type=jnp.float32)
    acc = acc + b_ref[0]
    if act == "sigmoid":
        acc = jax.nn.sigmoid(acc)
    elif act == "expclip":
        acc = jnp.exp(jnp.clip(acc, -15.0, 15.0))
    elif act == "softplus":
        m = jnp.minimum(acc, 20.0) * 1.4426950408889634
        sp = 0.6931471805599453 * jnp.log2(1.0 + jnp.exp2(m))
        acc = jnp.where(acc > 20.0, acc, sp)
    o_ref[...] = acc


def _run_head(d3t, wt, bstack, head, act):
    return pl.pallas_call(
        functools.partial(_head_body, act=act),
        grid=(3000 // BK,),
        in_specs=[pl.BlockSpec((BK, 1024), lambda i: (i, 0)),
                  pl.BlockSpec((1, BK, 1), lambda i: (head, i, 0)),
                  pl.BlockSpec(d3t.shape, lambda i: (0, 0))],
        out_specs=pl.BlockSpec((BK, N), lambda i: (i, 0)),
        out_shape=jax.ShapeDtypeStruct((3000, N), jnp.float32),
    )(wt, bstack, d3t)


GB = 512
NB = N // GB


def _gcn_chain_body(act_ref, adj_ref, ht_ref, wg1, wg2, wgz, wd1, wd2, wdz,
                    z_ref, zhat_ref, adjb_ref, fa_ref, fb_ref, zb_ref):
    s = pl.program_id(0)
    r = pl.program_id(1)
    act = act_ref[0]

    def relu_maybe(t):
        return jnp.where(act != 0, jnp.maximum(t, 0.0), t)

    rows = pl.ds(r * GB, GB)

    @pl.when(s == 0)
    def _():
        ab = _bf(adj_ref[...])
        adjb_ref[rows, :] = ab
        u = _bf(_dot0(ht_ref[...], wg1[...]))
        fa_ref[rows, :] = _bf(relu_maybe(
            jnp.dot(ab, u, preferred_element_type=jnp.float32)))

    @pl.when(s == 1)
    def _():
        u = _bf(jnp.dot(fa_ref[...], wg2[...],
                        preferred_element_type=jnp.float32))
        fb_ref[rows, :] = _bf(relu_maybe(
            jnp.dot(adjb_ref[rows, :], u, preferred_element_type=jnp.float32)))

    @pl.when(s == 2)
    def _():
        u = _bf(jnp.dot(fb_ref[...], wgz[...],
                        preferred_element_type=jnp.float32))
        t = jnp.dot(adjb_ref[rows, :], u, preferred_element_type=jnp.float32)
        z_ref[...] = t
        zb_ref[rows, :] = _bf(t)

    @pl.when(s == 3)
    def _():
        u = _bf(jnp.dot(zb_ref[...], wd1[...],
                        preferred_element_type=jnp.float32))
        fb_ref[rows, :] = _bf(relu_maybe(
            jnp.dot(adjb_ref[rows, :], u, preferred_element_type=jnp.float32)))

    @pl.when(s == 4)
    def _():
        u = _bf(jnp.dot(fb_ref[...], wd2[...],
                        preferred_element_type=jnp.float32))
        fa_ref[rows, :] = _bf(relu_maybe(
            jnp.dot(adjb_ref[rows, :], u, preferred_element_type=jnp.float32)))

    @pl.when(s == 5)
    def _():
        u = _bf(jnp.dot(fa_ref[...], wdz[...],
                        preferred_element_type=jnp.float32))
        zhat_ref[...] = jnp.dot(adjb_ref[rows, :], u,
                                preferred_element_type=jnp.float32)


def _run_gcn_chain(active_s, adj, ht, p):
    full = lambda a: pl.BlockSpec(a.shape, lambda s, r: (0,) * a.ndim)
    return pl.pallas_call(
        _gcn_chain_body,
        grid=(6, NB),
        in_specs=[pl.BlockSpec(memory_space=pltpu.SMEM),
                  pl.BlockSpec((GB, N),
                               lambda s, r: (jnp.where(s == 0, r, NB - 1), 0)),
                  full(ht),
                  full(p["Wg1"]), full(p["Wg2"]), full(p["Wgz"]),
                  full(p["Wd1"]), full(p["Wd2"]), full(p["Wdz"])],
        out_specs=(
            pl.BlockSpec((GB, 16),
                         lambda s, r: (jnp.where(s == 2, r, NB - 1), 0)),
            pl.BlockSpec((GB, 64),
                         lambda s, r: (jnp.where(s == 5, r, NB - 1), 0)),
        ),
        out_shape=(jax.ShapeDtypeStruct((N, 16), jnp.float32),
                   jax.ShapeDtypeStruct((N, 64), jnp.float32)),
        scratch_shapes=[pltpu.VMEM((N, N), jnp.bfloat16),
                        pltpu.VMEM((N, 256), jnp.bfloat16),
                        pltpu.VMEM((N, 64), jnp.bfloat16),
                        pltpu.VMEM((N, 16), jnp.bfloat16)],
    )(active_s, adj, ht, p["Wg1"], p["Wg2"], p["Wgz"],
      p["Wd1"], p["Wd2"], p["Wdz"])


def _adjhat_body(zr_ref, zc_ref, o_ref):
    acc = jax.lax.dot_general(zr_ref[...], zc_ref[...],
                              (((1,), (1,)), ((), ())),
                              preferred_element_type=jnp.float32)
    o_ref[...] = jax.nn.sigmoid(acc)


def _run_adjhat(z):
    bm, bn = 1024, 4096
    return pl.pallas_call(
        _adjhat_body,
        grid=(N // bm, N // bn),
        in_specs=[pl.BlockSpec((bm, 16), lambda i, j: (i, 0)),
                  pl.BlockSpec((bn, 16), lambda i, j: (j, 0))],
        out_specs=pl.BlockSpec((bm, bn), lambda i, j: (i, j)),
        out_shape=jax.ShapeDtypeStruct((N, N), jnp.float32),
    )(z, z)


def kernel(x, adj, active, params):
    p = params
    active_s = jnp.reshape(jnp.asarray(active, jnp.int32), (1,))

    trunk_w = [p[k] for k in
               ("W_en1", "W_en2", "W_en3", "W_h", "W_de1", "W_de2", "W_de3")]
    trunk_b = [jnp.reshape(p[k], (-1, 1)) for k in
               ("b_en1", "b_en2", "b_en3", "b_h", "b_de1", "b_de2", "b_de3")]
    ht, d3t = _run_trunk(x.T, trunk_w, trunk_b)
    h = ht.T

    bstack = jnp.reshape(
        jnp.stack([p["b_xhat"], p["b_pi"], p["b_mu"], p["b_theta"]]),
        (4, 3000, 1))
    x_hat = _run_head(d3t, p["W_xhat"].T, bstack, 0, "none").T
    pi = _run_head(d3t, p["W_pi"].T, bstack, 1, "sigmoid").T
    mu = _run_head(d3t, p["W_mu"].T, bstack, 2, "expclip").T
    theta = _run_head(d3t, p["W_theta"].T, bstack, 3, "softplus").T

    z, z_hat = _run_gcn_chain(active_s, adj, ht, p)
    adj_hat = _run_adjhat(z)

    return (x_hat, pi, mu, theta, z, adj_hat, z_hat, h)
